# baseline (speedup 1.0000x reference)
import functools
import math

import numpy as np
import jax
import jax.numpy as jnp
from jax import lax
from jax.experimental import pallas as pl
from jax.experimental.pallas import tpu as pltpu
from jax.experimental.pallas import tpu_sc as plsc

VOCAB = 100000
D_MODEL = 768
BATCH = 4
SEQ = 8192
TOK = BATCH * SEQ
NC, NS, LANES = 2, 16, 16
NW = NC * NS
PPW = SEQ // NW
KP = 32
NP = PPW // KP
TASKS = 2 * BATCH
NBUF = 3
SCALE = math.sqrt(D_MODEL)


def _pe_table():
    pos = np.arange(SEQ, dtype=np.float32)[:, None]
    i = np.arange(D_MODEL // 2, dtype=np.float32)[None, :]
    angle = pos / np.power(10000.0, (2.0 * i) / D_MODEL, dtype=np.float32)
    pe = np.zeros((SEQ, D_MODEL), dtype=np.float32)
    pe[:, 0::2] = np.sin(angle)
    pe[:, 1::2] = np.cos(angle)
    return pe


@functools.partial(
    pl.kernel,
    mesh=plsc.VectorSubcoreMesh(core_axis_name="c", subcore_axis_name="s"),
    out_type=jax.ShapeDtypeStruct((TOK, D_MODEL), jnp.float32),
    scratch_types=[
        pltpu.VMEM((BATCH, PPW), jnp.int32),
        pltpu.VMEM((KP, D_MODEL), jnp.float32),
        pltpu.VMEM((KP, D_MODEL), jnp.float32),
        pltpu.VMEM((KP, D_MODEL), jnp.float32),
        pltpu.VMEM((KP, D_MODEL), jnp.float32),
        pltpu.VMEM((KP, D_MODEL), jnp.float32),
        pltpu.SemaphoreType.DMA,
        pltpu.SemaphoreType.DMA,
        pltpu.SemaphoreType.DMA,
        pltpu.SemaphoreType.DMA,
        pltpu.SemaphoreType.DMA,
        pltpu.SemaphoreType.DMA,
        pltpu.SemaphoreType.DMA,
        pltpu.SemaphoreType.DMA,
        pltpu.SemaphoreType.DMA,
    ],
)
def _embed_sc(table_hbm, idx_hbm, pe_hbm, out_hbm,
              idx_all, rows0, rows1, rows2, pe0, pe1,
              sg0, sg1, sg2, ss0, ss1, ss2, spe0, spe1, sidx):
    rowsb, peb = [rows0, rows1, rows2], [pe0, pe1]
    sgb, ssb, speb = [sg0, sg1, sg2], [ss0, ss1, ss2], [spe0, spe1]

    wid = lax.axis_index("s") * NC + lax.axis_index("c")
    pbase = wid * PPW

    idx_cp = [
        pltpu.async_copy(idx_hbm.at[pl.ds(pl.multiple_of(b * SEQ + pbase, KP),
                                          PPW)],
                         idx_all.at[b], sidx)
        for b in range(BATCH)
    ]
    for c in idx_cp:
        c.wait()

    def outer(p2, carry):
        p = 2 * p2
        rel = [pl.multiple_of((p + pp) * KP, KP) for pp in range(2)]
        pos_off = [pl.multiple_of(pbase + (p + pp) * KP, KP) for pp in range(2)]
        pe_cp = [
            pltpu.async_copy(pe_hbm.at[pl.ds(pos_off[pp], KP)], peb[pp], speb[pp])
            for pp in range(2)
        ]

        def start_gather(t):
            pp, b = t // BATCH, t % BATCH
            off = pl.multiple_of(b * SEQ + pos_off[pp], KP)
            idx_ref = idx_all.at[b, pl.ds(rel[pp], KP)]
            return pltpu.async_copy(table_hbm.at[idx_ref], rowsb[t % NBUF],
                                    sgb[t % NBUF]), off

        gather = [None] * TASKS
        offs = [None] * TASKS
        store = [None] * TASKS
        gather[0], offs[0] = start_gather(0)
        gather[1], offs[1] = start_gather(1)
        for t in range(TASKS):
            pp = t // BATCH
            if t % BATCH == 0:
                pe_cp[pp].wait()
            if t + 2 < TASKS:
                if t >= 1:
                    store[t - 1].wait()
                gather[t + 2], offs[t + 2] = start_gather(t + 2)
            gather[t].wait()
            rv, pv = rowsb[t % NBUF], peb[pp]

            def row_body(r, rcarry):
                for l in range(D_MODEL // LANES):
                    sl = pl.ds(l * LANES, LANES)
                    rv[r, sl] = rv[r, sl] * SCALE + pv[r, sl]
                return rcarry

            lax.fori_loop(0, KP, row_body, 0)
            store[t] = pltpu.async_copy(rv, out_hbm.at[pl.ds(offs[t], KP)],
                                        ssb[t % NBUF])
        store[TASKS - 3].wait()
        store[TASKS - 2].wait()
        store[TASKS - 1].wait()
        return carry

    lax.fori_loop(0, NP // 2, outer, 0)


def kernel(x, table):
    idx = x.reshape(TOK).astype(jnp.int32)
    out = _embed_sc(table, idx, _pe_table())
    return out.reshape(BATCH, SEQ, D_MODEL)


# speedup vs baseline: 1.0244x; 1.0244x over previous
import functools
import math

import numpy as np
import jax
import jax.numpy as jnp
from jax import lax
from jax.experimental import pallas as pl
from jax.experimental.pallas import tpu as pltpu
from jax.experimental.pallas import tpu_sc as plsc

VOCAB = 100000
D_MODEL = 768
BATCH = 4
SEQ = 8192
TOK = BATCH * SEQ
NC, NS, LANES = 2, 16, 16
NW = NC * NS
PPW = SEQ // NW
KP = 32
NP = PPW // KP
TASKS = 2 * BATCH
NBUF = 3
SCALE = math.sqrt(D_MODEL)


def _pe_table():
    pos = np.arange(SEQ, dtype=np.float32)[:, None]
    i = np.arange(D_MODEL // 2, dtype=np.float32)[None, :]
    angle = pos / np.power(10000.0, (2.0 * i) / D_MODEL, dtype=np.float32)
    pe = np.zeros((SEQ, D_MODEL), dtype=np.float32)
    pe[:, 0::2] = np.sin(angle)
    pe[:, 1::2] = np.cos(angle)
    return pe


@functools.partial(
    pl.kernel,
    mesh=plsc.VectorSubcoreMesh(core_axis_name="c", subcore_axis_name="s"),
    out_type=jax.ShapeDtypeStruct((TOK, D_MODEL), jnp.float32),
    scratch_types=[
        pltpu.VMEM((BATCH, PPW), jnp.int32),
        pltpu.VMEM((KP, D_MODEL), jnp.float32),
        pltpu.VMEM((KP, D_MODEL), jnp.float32),
        pltpu.VMEM((KP, D_MODEL), jnp.float32),
        pltpu.VMEM((KP, D_MODEL), jnp.float32),
        pltpu.VMEM((KP, D_MODEL), jnp.float32),
        pltpu.SemaphoreType.DMA,
        pltpu.SemaphoreType.DMA,
        pltpu.SemaphoreType.DMA,
        pltpu.SemaphoreType.DMA,
        pltpu.SemaphoreType.DMA,
        pltpu.SemaphoreType.DMA,
        pltpu.SemaphoreType.DMA,
        pltpu.SemaphoreType.DMA,
        pltpu.SemaphoreType.DMA,
    ],
)
def _embed_sc(table_hbm, idx_hbm, pe_hbm, out_hbm,
              idx_all, rows0, rows1, rows2, pe0, pe1,
              sg0, sg1, sg2, ss0, ss1, ss2, spe0, spe1, sidx):
    rowsb, peb = [rows0, rows1, rows2], [pe0, pe1]
    sgb, ssb, speb = [sg0, sg1, sg2], [ss0, ss1, ss2], [spe0, spe1]

    wid = lax.axis_index("s") * NC + lax.axis_index("c")
    pbase = wid * PPW

    idx_cp = [
        pltpu.async_copy(idx_hbm.at[pl.ds(pl.multiple_of(b * SEQ + pbase, KP),
                                          PPW)],
                         idx_all.at[b], sidx)
        for b in range(BATCH)
    ]
    for c in idx_cp:
        c.wait()

    def pe_start(c):
        off = pl.multiple_of(pbase + c * KP, KP)
        return pltpu.async_copy(pe_hbm.at[pl.ds(off, KP)], peb[c % 2],
                                speb[c % 2])

    def start_gather(t):
        c, b = t // BATCH, t % BATCH
        off = pl.multiple_of(b * SEQ + pbase + c * KP, KP)
        idx_ref = idx_all.at[b, pl.ds(c * KP, KP)]
        return pltpu.async_copy(table_hbm.at[idx_ref], rowsb[t % NBUF],
                                sgb[t % NBUF]), off

    ntasks = NP * BATCH
    pe_cp = [None] * NP
    pe_cp[0] = pe_start(0)
    pe_cp[1] = pe_start(1)
    gather = [None] * ntasks
    offs = [None] * ntasks
    store = [None] * ntasks
    gather[0], offs[0] = start_gather(0)
    gather[1], offs[1] = start_gather(1)
    for t in range(ntasks):
        c = t // BATCH
        if t % BATCH == 0:
            pe_cp[c].wait()
        if t + 2 < ntasks:
            if t >= 1:
                store[t - 1].wait()
            gather[t + 2], offs[t + 2] = start_gather(t + 2)
        gather[t].wait()
        rv, pv = rowsb[t % NBUF], peb[c % 2]

        def row_body(r, rcarry):
            for l in range(D_MODEL // LANES):
                sl = pl.ds(l * LANES, LANES)
                rv[r, sl] = rv[r, sl] * SCALE + pv[r, sl]
            return rcarry

        lax.fori_loop(0, KP, row_body, 0)
        store[t] = pltpu.async_copy(rv, out_hbm.at[pl.ds(offs[t], KP)],
                                    ssb[t % NBUF])
        if t % BATCH == BATCH - 1 and c + 2 < NP:
            pe_cp[c + 2] = pe_start(c + 2)
    store[ntasks - 3].wait()
    store[ntasks - 2].wait()
    store[ntasks - 1].wait()


def kernel(x, table):
    idx = x.reshape(TOK).astype(jnp.int32)
    out = _embed_sc(table, idx, _pe_table())
    return out.reshape(BATCH, SEQ, D_MODEL)
